# baseline (speedup 1.0000x reference)
import functools

import jax
import jax.numpy as jnp
from jax import lax
from jax.experimental import pallas as pl
from jax.experimental.pallas import tpu as pltpu
from jax.experimental.pallas import tpu_sc as plsc

N = 10000
E = 320000
NP = 10112
NC, NS = 1, 16
NW = NC * NS
CB = 128
NB = 8
NCHUNK = 160
NOUT = NCHUNK // NB
EP = NW * CB * NCHUNK
RPT = NP // NS
PAD_DST = N + 8

_mesh = plsc.VectorSubcoreMesh(core_axis_name="c", subcore_axis_name="s",
                               num_cores=1)


def _make_deg_kernel():
    @functools.partial(
        pl.kernel,
        out_type=jax.ShapeDtypeStruct((NC, NP, 16), jnp.float32),
        mesh=_mesh,
        scratch_types=[
            pltpu.VMEM((NCHUNK, CB), jnp.int32),
            pltpu.VMEM((CB, 16), jnp.float32),
            pltpu.VMEM_SHARED((NP, 16), jnp.float32),
        ] + [pltpu.SemaphoreType.DMA] * NB,
        compiler_params=pltpu.CompilerParams(use_tc_tiling_on_sc=False),
    )
    def k(dst_hbm, ones_hbm, zeros_hbm, out_hbm, didx, rows, acc, *sem_s):
        c = lax.axis_index("c")
        s = lax.axis_index("s")
        wid = s * NC + c
        pltpu.sync_copy(zeros_hbm.at[pl.ds(s * RPT, RPT)],
                        acc.at[pl.ds(s * RPT, RPT)])
        pltpu.sync_copy(dst_hbm.at[wid], didx)
        pltpu.sync_copy(ones_hbm, rows)
        plsc.subcore_barrier()

        for b in range(NB):
            pltpu.async_copy(rows, acc.at[didx.at[b]], sem_s[b], add=True)

        def body(t, carry):
            for b in range(NB):
                j = t * NB + b
                pltpu.make_async_copy(rows, acc.at[didx.at[j]],
                                      sem_s[b]).wait()
                jn = lax.rem(j + NB, NCHUNK)
                pltpu.async_copy(rows, acc.at[didx.at[jn]], sem_s[b],
                                 add=True)
            return carry

        lax.fori_loop(0, NOUT - 1, body, 0)
        for b in range(NB):
            pltpu.make_async_copy(rows, acc.at[didx.at[b]], sem_s[b]).wait()
        plsc.subcore_barrier()
        pltpu.sync_copy(acc.at[pl.ds(s * RPT, RPT)],
                        out_hbm.at[c, pl.ds(s * RPT, RPT)])

    return k


def _make_edge_kernel(D):
    @functools.partial(
        pl.kernel,
        out_type=jax.ShapeDtypeStruct((NC, NP, D), jnp.float32),
        mesh=_mesh,
        scratch_types=[
            pltpu.VMEM((NCHUNK, CB), jnp.int32),
            pltpu.VMEM((NCHUNK, CB), jnp.int32),
            pltpu.VMEM_SHARED((NP, D), jnp.float32),
        ] + [pltpu.VMEM((CB, D), jnp.float32)] * NB
          + [pltpu.SemaphoreType.DMA] * (2 * NB),
        compiler_params=pltpu.CompilerParams(use_tc_tiling_on_sc=False),
    )
    def k(src_hbm, dst_hbm, g_hbm, zeros_hbm, out_hbm, sidx, didx, acc,
          *rest):
        bufs = rest[:NB]
        sem_g = rest[NB:2 * NB]
        sem_s = rest[2 * NB:]
        c = lax.axis_index("c")
        s = lax.axis_index("s")
        wid = s * NC + c
        pltpu.sync_copy(zeros_hbm.at[pl.ds(s * RPT, RPT)],
                        acc.at[pl.ds(s * RPT, RPT)])
        pltpu.sync_copy(src_hbm.at[wid], sidx)
        pltpu.sync_copy(dst_hbm.at[wid], didx)
        plsc.subcore_barrier()

        for b in range(NB):
            pltpu.async_copy(g_hbm.at[sidx.at[b]], bufs[b], sem_g[b])

        def body(t, carry):
            for b in range(NB):
                j = t * NB + b
                pltpu.make_async_copy(g_hbm.at[sidx.at[j]], bufs[b],
                                      sem_g[b]).wait()
                pltpu.async_copy(bufs[b], acc.at[didx.at[j]], sem_s[b],
                                 add=True)
            for b in range(NB):
                j = t * NB + b
                jn = lax.rem(j + NB, NCHUNK)
                pltpu.make_async_copy(bufs[b], acc.at[didx.at[j]],
                                      sem_s[b]).wait()
                pltpu.async_copy(g_hbm.at[sidx.at[jn]], bufs[b], sem_g[b])
            return carry

        lax.fori_loop(0, NOUT, body, 0)
        for b in range(NB):
            pltpu.make_async_copy(g_hbm.at[sidx.at[b]], bufs[b],
                                  sem_g[b]).wait()
        plsc.subcore_barrier()
        pltpu.sync_copy(acc.at[pl.ds(s * RPT, RPT)],
                        out_hbm.at[c, pl.ds(s * RPT, RPT)])

    return k


_deg_kernel = _make_deg_kernel()
_edge16_kernel = _make_edge_kernel(16)


_BR = 1000
_GRID = N // _BR


def _tc1_body(degacc_ref, x_ref, w1_ref, dinv_ref, g1_ref):
    deg = degacc_ref[0, :, 0:1] + degacc_ref[1, :, 0:1] + 1.0
    dinv = lax.rsqrt(deg)
    dinv_ref[...] = dinv
    h = jnp.dot(x_ref[...], w1_ref[...], preferred_element_type=jnp.float32)
    g1_ref[...] = h * dinv


def _tc2_body(acc_ref, g1_ref, dinv_ref, b1_ref, g2_ref):
    dinv = dinv_ref[...]
    h1 = dinv * (acc_ref[0] + acc_ref[1] + g1_ref[...]) + b1_ref[...]
    h1 = jnp.maximum(h1, 0.0)
    g2_ref[...] = h1 * dinv


def _tc3_body(acc_ref, g2_ref, dinv_ref, w2_ref, b2_ref, fw1_ref, fb1_ref,
              fw2_ref, fb2_ref, fw3_ref, fb3_ref, out_ref):
    dinv = dinv_ref[...]
    agg = dinv * (acc_ref[0] + acc_ref[1] + g2_ref[...])
    h = jnp.dot(agg, w2_ref[...], preferred_element_type=jnp.float32)
    h = jnp.maximum(h + b2_ref[...], 0.0)
    h = jnp.dot(h, fw1_ref[...], preferred_element_type=jnp.float32)
    h = jnp.maximum(h + fb1_ref[...], 0.0)
    h = jnp.dot(h, fw2_ref[...], preferred_element_type=jnp.float32)
    h = jnp.maximum(h + fb2_ref[...], 0.0)
    h = jnp.dot(h, fw3_ref[...], preferred_element_type=jnp.float32)
    out_ref[...] = h + fb3_ref[...]


def _row_spec(d):
    return pl.BlockSpec((_BR, d), lambda i: (i, 0))


def _acc_spec(d):
    return pl.BlockSpec((NC, _BR, d), lambda i: (0, i, 0))


def _full_spec(shape):
    return pl.BlockSpec(shape, lambda i: tuple(0 for _ in shape))


def kernel(x, edge_index, W1, b1, W2, b2, fW1, fb1, fW2, fb2, fW3, fb3):
    src = edge_index[0].astype(jnp.int32)
    dst = edge_index[1].astype(jnp.int32)
    pad = EP - E
    src = jnp.concatenate([src, jnp.zeros((pad,), jnp.int32)])
    dst = jnp.concatenate([dst, jnp.full((pad,), PAD_DST, jnp.int32)])
    src = src.reshape(NW, NCHUNK, CB)
    dst = dst.reshape(NW, NCHUNK, CB)

    ones16 = jnp.ones((CB, 16), jnp.float32)
    z16 = jnp.zeros((NP, 16), jnp.float32)

    degacc = _deg_kernel(dst, ones16, z16)

    dinv, g1 = pl.pallas_call(
        _tc1_body,
        grid=(_GRID,),
        in_specs=[_acc_spec(16), _row_spec(128), _full_spec((128, 16))],
        out_specs=[_row_spec(1), _row_spec(16)],
        out_shape=[
            jax.ShapeDtypeStruct((N, 1), jnp.float32),
            jax.ShapeDtypeStruct((N, 16), jnp.float32),
        ],
    )(degacc, x, W1)

    acc1 = _edge16_kernel(src, dst, g1, z16)

    g2 = pl.pallas_call(
        _tc2_body,
        grid=(_GRID,),
        in_specs=[_acc_spec(16), _row_spec(16), _row_spec(1),
                  _full_spec((1, 16))],
        out_specs=_row_spec(16),
        out_shape=jax.ShapeDtypeStruct((N, 16), jnp.float32),
    )(acc1, g1, dinv, b1.reshape(1, 16))

    acc2 = _edge16_kernel(src, dst, g2, z16)

    out = pl.pallas_call(
        _tc3_body,
        grid=(_GRID,),
        in_specs=[_acc_spec(16), _row_spec(16), _row_spec(1),
                  _full_spec((16, 32)),
                  _full_spec((1, 32)), _full_spec((32, 64)),
                  _full_spec((1, 64)), _full_spec((64, 32)),
                  _full_spec((1, 32)), _full_spec((32, 40)),
                  _full_spec((1, 40))],
        out_specs=_row_spec(40),
        out_shape=jax.ShapeDtypeStruct((N, 40), jnp.float32),
    )(acc2, g2, dinv, W2, b2.reshape(1, 32), fW1, fb1.reshape(1, 64), fW2,
      fb2.reshape(1, 32), fW3, fb3.reshape(1, 40))

    return out


# speedup vs baseline: 1.0965x; 1.0965x over previous
import functools

import jax
import jax.numpy as jnp
from jax import lax
from jax.experimental import pallas as pl
from jax.experimental.pallas import tpu as pltpu
from jax.experimental.pallas import tpu_sc as plsc

N = 10000
E = 320000
NP = 10112
NC, NS = 2, 16
NW = NC * NS
CB = 128
NB = 8
NCHUNK = 80
NOUT = NCHUNK // NB
EP = NW * CB * NCHUNK
RPT = NP // NS
PAD_DST = N + 8

_mesh = plsc.VectorSubcoreMesh(core_axis_name="c", subcore_axis_name="s")


def _make_deg_kernel():
    @functools.partial(
        pl.kernel,
        out_type=jax.ShapeDtypeStruct((NC, NP, 16), jnp.float32),
        mesh=_mesh,
        scratch_types=[
            pltpu.VMEM((NCHUNK, CB), jnp.int32),
            pltpu.VMEM((CB, 16), jnp.float32),
            pltpu.VMEM_SHARED((NP, 16), jnp.float32),
        ] + [pltpu.SemaphoreType.DMA] * NB,
        compiler_params=pltpu.CompilerParams(use_tc_tiling_on_sc=False),
    )
    def k(dst_hbm, ones_hbm, zeros_hbm, out_hbm, didx, rows, acc, *sem_s):
        c = lax.axis_index("c")
        s = lax.axis_index("s")
        wid = s * NC + c
        pltpu.sync_copy(zeros_hbm.at[pl.ds(s * RPT, RPT)],
                        acc.at[pl.ds(s * RPT, RPT)])
        pltpu.sync_copy(dst_hbm.at[wid], didx)
        pltpu.sync_copy(ones_hbm, rows)
        plsc.subcore_barrier()

        def body(j, carry):
            pltpu.sync_copy(rows, acc.at[didx.at[j]], add=True)
            return carry

        lax.fori_loop(0, NCHUNK, body, 0)
        plsc.subcore_barrier()
        pltpu.sync_copy(acc.at[pl.ds(s * RPT, RPT)],
                        out_hbm.at[c, pl.ds(s * RPT, RPT)])

    return k


def _make_edge_kernel(D):
    @functools.partial(
        pl.kernel,
        out_type=jax.ShapeDtypeStruct((NC, NP, D), jnp.float32),
        mesh=_mesh,
        scratch_types=[
            pltpu.VMEM((NCHUNK, CB), jnp.int32),
            pltpu.VMEM((NCHUNK, CB), jnp.int32),
            pltpu.VMEM_SHARED((NP, D), jnp.float32),
        ] + [pltpu.VMEM((CB, D), jnp.float32)] * NB
          + [pltpu.SemaphoreType.DMA] * (2 * NB),
        compiler_params=pltpu.CompilerParams(use_tc_tiling_on_sc=False),
    )
    def k(src_hbm, dst_hbm, g_hbm, zeros_hbm, out_hbm, sidx, didx, acc,
          *rest):
        bufs = rest[:NB]
        sem_g = rest[NB:2 * NB]
        sem_s = rest[2 * NB:]
        c = lax.axis_index("c")
        s = lax.axis_index("s")
        wid = s * NC + c
        pltpu.sync_copy(zeros_hbm.at[pl.ds(s * RPT, RPT)],
                        acc.at[pl.ds(s * RPT, RPT)])
        pltpu.sync_copy(src_hbm.at[wid], sidx)
        pltpu.sync_copy(dst_hbm.at[wid], didx)
        plsc.subcore_barrier()

        for b in range(NB):
            pltpu.async_copy(g_hbm.at[sidx.at[b]], bufs[b], sem_g[b])

        def body(t, carry):
            for b in range(NB):
                j = t * NB + b
                pltpu.make_async_copy(g_hbm.at[sidx.at[j]], bufs[b],
                                      sem_g[b]).wait()
                pltpu.async_copy(bufs[b], acc.at[didx.at[j]], sem_s[b],
                                 add=True)
            for b in range(NB):
                j = t * NB + b
                jn = lax.rem(j + NB, NCHUNK)
                pltpu.make_async_copy(bufs[b], acc.at[didx.at[j]],
                                      sem_s[b]).wait()
                pltpu.async_copy(g_hbm.at[sidx.at[jn]], bufs[b], sem_g[b])
            return carry

        lax.fori_loop(0, NOUT, body, 0)
        for b in range(NB):
            pltpu.make_async_copy(g_hbm.at[sidx.at[b]], bufs[b],
                                  sem_g[b]).wait()
        plsc.subcore_barrier()
        pltpu.sync_copy(acc.at[pl.ds(s * RPT, RPT)],
                        out_hbm.at[c, pl.ds(s * RPT, RPT)])

    return k


_deg_kernel = _make_deg_kernel()
_edge16_kernel = _make_edge_kernel(16)


_BR = 1000
_GRID = N // _BR


def _tc1_body(degacc_ref, x_ref, w1_ref, dinv_ref, g1_ref):
    deg = degacc_ref[0, :, 0:1] + degacc_ref[1, :, 0:1] + 1.0
    dinv = lax.rsqrt(deg)
    dinv_ref[...] = dinv
    h = jnp.dot(x_ref[...], w1_ref[...], preferred_element_type=jnp.float32)
    g1_ref[...] = h * dinv


def _tc2_body(acc_ref, g1_ref, dinv_ref, b1_ref, g2_ref):
    dinv = dinv_ref[...]
    h1 = dinv * (acc_ref[0] + acc_ref[1] + g1_ref[...]) + b1_ref[...]
    h1 = jnp.maximum(h1, 0.0)
    g2_ref[...] = h1 * dinv


def _tc3_body(acc_ref, g2_ref, dinv_ref, w2_ref, b2_ref, fw1_ref, fb1_ref,
              fw2_ref, fb2_ref, fw3_ref, fb3_ref, out_ref):
    dinv = dinv_ref[...]
    agg = dinv * (acc_ref[0] + acc_ref[1] + g2_ref[...])
    h = jnp.dot(agg, w2_ref[...], preferred_element_type=jnp.float32)
    h = jnp.maximum(h + b2_ref[...], 0.0)
    h = jnp.dot(h, fw1_ref[...], preferred_element_type=jnp.float32)
    h = jnp.maximum(h + fb1_ref[...], 0.0)
    h = jnp.dot(h, fw2_ref[...], preferred_element_type=jnp.float32)
    h = jnp.maximum(h + fb2_ref[...], 0.0)
    h = jnp.dot(h, fw3_ref[...], preferred_element_type=jnp.float32)
    out_ref[...] = h + fb3_ref[...]


def _row_spec(d):
    return pl.BlockSpec((_BR, d), lambda i: (i, 0))


def _acc_spec(d):
    return pl.BlockSpec((NC, _BR, d), lambda i: (0, i, 0))


def _full_spec(shape):
    return pl.BlockSpec(shape, lambda i: tuple(0 for _ in shape))


def kernel(x, edge_index, W1, b1, W2, b2, fW1, fb1, fW2, fb2, fW3, fb3):
    src = edge_index[0].astype(jnp.int32)
    dst = edge_index[1].astype(jnp.int32)
    pad = EP - E
    src = jnp.concatenate([src, jnp.zeros((pad,), jnp.int32)])
    dst = jnp.concatenate([dst, jnp.full((pad,), PAD_DST, jnp.int32)])
    src = src.reshape(NW, NCHUNK, CB)
    dst = dst.reshape(NW, NCHUNK, CB)

    ones16 = jnp.ones((CB, 16), jnp.float32)
    z16 = jnp.zeros((NP, 16), jnp.float32)

    degacc = _deg_kernel(dst, ones16, z16)

    dinv, g1 = pl.pallas_call(
        _tc1_body,
        grid=(_GRID,),
        in_specs=[_acc_spec(16), _row_spec(128), _full_spec((128, 16))],
        out_specs=[_row_spec(1), _row_spec(16)],
        out_shape=[
            jax.ShapeDtypeStruct((N, 1), jnp.float32),
            jax.ShapeDtypeStruct((N, 16), jnp.float32),
        ],
    )(degacc, x, W1)

    acc1 = _edge16_kernel(src, dst, g1, z16)

    g2 = pl.pallas_call(
        _tc2_body,
        grid=(_GRID,),
        in_specs=[_acc_spec(16), _row_spec(16), _row_spec(1),
                  _full_spec((1, 16))],
        out_specs=_row_spec(16),
        out_shape=jax.ShapeDtypeStruct((N, 16), jnp.float32),
    )(acc1, g1, dinv, b1.reshape(1, 16))

    acc2 = _edge16_kernel(src, dst, g2, z16)

    out = pl.pallas_call(
        _tc3_body,
        grid=(_GRID,),
        in_specs=[_acc_spec(16), _row_spec(16), _row_spec(1),
                  _full_spec((16, 32)),
                  _full_spec((1, 32)), _full_spec((32, 64)),
                  _full_spec((1, 64)), _full_spec((64, 32)),
                  _full_spec((1, 32)), _full_spec((32, 40)),
                  _full_spec((1, 40))],
        out_specs=_row_spec(40),
        out_shape=jax.ShapeDtypeStruct((N, 40), jnp.float32),
    )(acc2, g2, dinv, W2, b2.reshape(1, 32), fW1, fb1.reshape(1, 64), fW2,
      fb2.reshape(1, 32), fW3, fb3.reshape(1, 40))

    return out


# speedup vs baseline: 1.0973x; 1.0008x over previous
import functools

import jax
import jax.numpy as jnp
from jax import lax
from jax.experimental import pallas as pl
from jax.experimental.pallas import tpu as pltpu
from jax.experimental.pallas import tpu_sc as plsc

N = 10000
E = 320000
NP = 10112
NC, NS = 2, 16
NW = NC * NS
CB = 128
NB = 8
NCHUNK = 80
NOUT = NCHUNK // NB
EP = NW * CB * NCHUNK
RPT = NP // NS
PAD_DST = N + 8

_mesh = plsc.VectorSubcoreMesh(core_axis_name="c", subcore_axis_name="s")


def _make_deg_kernel():
    @functools.partial(
        pl.kernel,
        out_type=jax.ShapeDtypeStruct((NC, NP, 16), jnp.float32),
        mesh=_mesh,
        scratch_types=[
            pltpu.VMEM((NCHUNK, CB), jnp.int32),
            pltpu.VMEM((CB, 16), jnp.float32),
            pltpu.VMEM_SHARED((NP, 16), jnp.float32),
        ],
        compiler_params=pltpu.CompilerParams(use_tc_tiling_on_sc=False),
    )
    def k(dst_hbm, ones_hbm, zeros_hbm, out_hbm, didx, rows, acc):
        c = lax.axis_index("c")
        s = lax.axis_index("s")
        wid = s * NC + c
        pltpu.sync_copy(zeros_hbm.at[pl.ds(s * RPT, RPT)],
                        acc.at[pl.ds(s * RPT, RPT)])
        pltpu.sync_copy(dst_hbm.at[wid], didx)
        pltpu.sync_copy(ones_hbm, rows)
        plsc.subcore_barrier()

        def body(j, carry):
            pltpu.sync_copy(rows, acc.at[didx.at[j]], add=True)
            return carry

        lax.fori_loop(0, NCHUNK, body, 0)
        plsc.subcore_barrier()
        pltpu.sync_copy(acc.at[pl.ds(s * RPT, RPT)],
                        out_hbm.at[c, pl.ds(s * RPT, RPT)])

    return k


def _make_edge_kernel(D):
    @functools.partial(
        pl.kernel,
        out_type=jax.ShapeDtypeStruct((NC, NP, D), jnp.float32),
        mesh=_mesh,
        scratch_types=[
            pltpu.VMEM((NCHUNK, CB), jnp.int32),
            pltpu.VMEM((NCHUNK, CB), jnp.int32),
            pltpu.VMEM_SHARED((NP, D), jnp.float32),
        ] + [pltpu.VMEM((CB, D), jnp.float32)] * NB
          + [pltpu.SemaphoreType.DMA] * (2 * NB),
        compiler_params=pltpu.CompilerParams(use_tc_tiling_on_sc=False),
    )
    def k(src_hbm, dst_hbm, g_hbm, zeros_hbm, out_hbm, sidx, didx, acc,
          *rest):
        bufs = rest[:NB]
        sem_g = rest[NB:2 * NB]
        sem_s = rest[2 * NB:]
        c = lax.axis_index("c")
        s = lax.axis_index("s")
        wid = s * NC + c
        pltpu.sync_copy(zeros_hbm.at[pl.ds(s * RPT, RPT)],
                        acc.at[pl.ds(s * RPT, RPT)])
        pltpu.sync_copy(src_hbm.at[wid], sidx)
        pltpu.sync_copy(dst_hbm.at[wid], didx)
        plsc.subcore_barrier()

        for b in range(NB):
            pltpu.async_copy(g_hbm.at[sidx.at[b]], bufs[b], sem_g[b])

        def body(t, carry):
            for b in range(NB):
                j = t * NB + b
                pltpu.make_async_copy(g_hbm.at[sidx.at[j]], bufs[b],
                                      sem_g[b]).wait()
                pltpu.async_copy(bufs[b], acc.at[didx.at[j]], sem_s[b],
                                 add=True)
            for b in range(NB):
                j = t * NB + b
                jn = lax.rem(j + NB, NCHUNK)
                pltpu.make_async_copy(bufs[b], acc.at[didx.at[j]],
                                      sem_s[b]).wait()
                pltpu.async_copy(g_hbm.at[sidx.at[jn]], bufs[b], sem_g[b])
            return carry

        lax.fori_loop(0, NOUT, body, 0)
        for b in range(NB):
            pltpu.make_async_copy(g_hbm.at[sidx.at[b]], bufs[b],
                                  sem_g[b]).wait()
        plsc.subcore_barrier()
        pltpu.sync_copy(acc.at[pl.ds(s * RPT, RPT)],
                        out_hbm.at[c, pl.ds(s * RPT, RPT)])

    return k


_deg_kernel = _make_deg_kernel()
_edge16_kernel = _make_edge_kernel(16)


_BR = 1000
_GRID = N // _BR


def _tc1_body(degacc_ref, x_ref, w1_ref, dinv_ref, g1_ref):
    deg = degacc_ref[0, :, 0:1] + degacc_ref[1, :, 0:1] + 1.0
    dinv = lax.rsqrt(deg)
    dinv_ref[...] = dinv
    h = jnp.dot(x_ref[...], w1_ref[...], preferred_element_type=jnp.float32)
    g1_ref[...] = h * dinv


def _tc2_body(acc_ref, g1_ref, dinv_ref, b1_ref, g2_ref):
    dinv = dinv_ref[...]
    h1 = dinv * (acc_ref[0] + acc_ref[1] + g1_ref[...]) + b1_ref[...]
    h1 = jnp.maximum(h1, 0.0)
    g2_ref[...] = h1 * dinv


def _tc3_body(acc_ref, g2_ref, dinv_ref, w2_ref, b2_ref, fw1_ref, fb1_ref,
              fw2_ref, fb2_ref, fw3_ref, fb3_ref, out_ref):
    dinv = dinv_ref[...]
    agg = dinv * (acc_ref[0] + acc_ref[1] + g2_ref[...])
    h = jnp.dot(agg, w2_ref[...], preferred_element_type=jnp.float32)
    h = jnp.maximum(h + b2_ref[...], 0.0)
    h = jnp.dot(h, fw1_ref[...], preferred_element_type=jnp.float32)
    h = jnp.maximum(h + fb1_ref[...], 0.0)
    h = jnp.dot(h, fw2_ref[...], preferred_element_type=jnp.float32)
    h = jnp.maximum(h + fb2_ref[...], 0.0)
    h = jnp.dot(h, fw3_ref[...], preferred_element_type=jnp.float32)
    out_ref[...] = h + fb3_ref[...]


def _row_spec(d):
    return pl.BlockSpec((_BR, d), lambda i: (i, 0))


def _acc_spec(d):
    return pl.BlockSpec((NC, _BR, d), lambda i: (0, i, 0))


def _full_spec(shape):
    return pl.BlockSpec(shape, lambda i: tuple(0 for _ in shape))


def kernel(x, edge_index, W1, b1, W2, b2, fW1, fb1, fW2, fb2, fW3, fb3):
    src = edge_index[0].astype(jnp.int32)
    dst = edge_index[1].astype(jnp.int32)
    pad = EP - E
    src = jnp.concatenate([src, jnp.zeros((pad,), jnp.int32)])
    dst = jnp.concatenate([dst, jnp.full((pad,), PAD_DST, jnp.int32)])
    src = src.reshape(NW, NCHUNK, CB)
    dst = dst.reshape(NW, NCHUNK, CB)

    ones16 = jnp.ones((CB, 16), jnp.float32)
    z16 = jnp.zeros((NP, 16), jnp.float32)

    degacc = _deg_kernel(dst, ones16, z16)

    dinv, g1 = pl.pallas_call(
        _tc1_body,
        grid=(_GRID,),
        in_specs=[_acc_spec(16), _row_spec(128), _full_spec((128, 16))],
        out_specs=[_row_spec(1), _row_spec(16)],
        out_shape=[
            jax.ShapeDtypeStruct((N, 1), jnp.float32),
            jax.ShapeDtypeStruct((N, 16), jnp.float32),
        ],
    )(degacc, x, W1)

    acc1 = _edge16_kernel(src, dst, g1, z16)

    g2 = pl.pallas_call(
        _tc2_body,
        grid=(_GRID,),
        in_specs=[_acc_spec(16), _row_spec(16), _row_spec(1),
                  _full_spec((1, 16))],
        out_specs=_row_spec(16),
        out_shape=jax.ShapeDtypeStruct((N, 16), jnp.float32),
    )(acc1, g1, dinv, b1.reshape(1, 16))

    acc2 = _edge16_kernel(src, dst, g2, z16)

    out = pl.pallas_call(
        _tc3_body,
        grid=(_GRID,),
        in_specs=[_acc_spec(16), _row_spec(16), _row_spec(1),
                  _full_spec((16, 32)),
                  _full_spec((1, 32)), _full_spec((32, 64)),
                  _full_spec((1, 64)), _full_spec((64, 32)),
                  _full_spec((1, 32)), _full_spec((32, 40)),
                  _full_spec((1, 40))],
        out_specs=_row_spec(40),
        out_shape=jax.ShapeDtypeStruct((N, 40), jnp.float32),
    )(acc2, g2, dinv, W2, b2.reshape(1, 32), fW1, fb1.reshape(1, 64), fW2,
      fb2.reshape(1, 32), fW3, fb3.reshape(1, 40))

    return out
